# baseline (speedup 1.0000x reference)
import jax
import jax.numpy as jnp
from jax.experimental import pallas as pl
from jax.experimental.pallas import tpu as pltpu

_ROWS = 100000
_DIM = 128
_SIZES = [1000, 1000, 2000, 4000] + [6000] * 14 + [4000, 2000, 1000, 1000]
_STARTS = [sum(_SIZES[:i]) for i in range(len(_SIZES))]
_NCHUNKS = len(_SIZES)
_AHEAD = 3


def _copy_body(in_hbm, out_hbm, buf, in_sems, out_sems):
    def copy_in(c):
        return pltpu.make_async_copy(
            in_hbm.at[pl.ds(_STARTS[c], _SIZES[c])],
            buf.at[pl.ds(_STARTS[c], _SIZES[c])],
            in_sems.at[c])

    def copy_out(c):
        return pltpu.make_async_copy(
            buf.at[pl.ds(_STARTS[c], _SIZES[c])],
            out_hbm.at[pl.ds(_STARTS[c], _SIZES[c])],
            out_sems.at[c])

    for c in range(_AHEAD):
        copy_in(c).start()
    for c in range(_NCHUNKS):
        copy_in(c).wait()
        copy_out(c).start()
        if c + _AHEAD < _NCHUNKS:
            copy_in(c + _AHEAD).start()
    for c in range(_NCHUNKS):
        copy_out(c).wait()


def kernel(weight):
    return pl.pallas_call(
        _copy_body,
        in_specs=[pl.BlockSpec(memory_space=pl.ANY)],
        out_specs=pl.BlockSpec(memory_space=pl.ANY),
        out_shape=jax.ShapeDtypeStruct((_ROWS, _DIM), jnp.float32),
        scratch_shapes=[
            pltpu.VMEM((_ROWS, _DIM), jnp.float32),
            pltpu.SemaphoreType.DMA((_NCHUNKS,)),
            pltpu.SemaphoreType.DMA((_NCHUNKS,)),
        ],
    )(weight)
